# baseline (speedup 1.0000x reference)
import functools
import math

import jax
import jax.numpy as jnp
from jax import lax
from jax.experimental import pallas as pl
from jax.experimental.pallas import tpu as pltpu
from jax.experimental.pallas import tpu_sc as plsc

D_MODEL = 64
SCALE = math.sqrt(D_MODEL)

_NC = 2
_NS = 16
_NW = _NC * _NS

_LANES = 128
_SUB = 8
_NBUF = 4
_UNROLL = 2


@functools.lru_cache(maxsize=None)
def _build(n_seq: int, n_batch: int, vocab: int, d_model: int):
    assert d_model == D_MODEL and n_batch % _LANES == 0 and n_seq % _SUB == 0
    n_bb = n_batch // _LANES
    n_units = n_seq * n_bb
    units_per_w = n_units // _NW
    assert n_units % (_NW * _NBUF) == 0
    n_jb = d_model // _SUB
    tile_words = _SUB * _LANES

    def body(x_hbm, table_hbm, out_hbm, idx_v, rows, tiles, gsems, wsems):
        wid = lax.axis_index("s") * _NC + lax.axis_index("c")
        u0 = wid * units_per_w

        pltpu.sync_copy(x_hbm.at[pl.ds(u0, units_per_w)], idx_v)

        iota = lax.iota(jnp.int32, 16)
        jbase = [iota * _LANES + 2048 * k for k in range(d_model // 16)]

        def fire_gather(t, slot):
            pltpu.async_copy(table_hbm.at[idx_v.at[t]], rows[slot],
                             gsems[slot])

        def drain_gather(t, slot):
            pltpu.make_async_copy(table_hbm.at[idx_v.at[t]], rows[slot],
                                  gsems[slot]).wait()

        def unit_coords(t):
            u = u0 + t
            s = (u >> 8) * _SUB + (u & (_SUB - 1))
            bb = (u >> 3) & (n_bb - 1)
            return s, bb

        def outer(o, carry):
            for b in range(_NBUF):
                t = o * _NBUF + b
                drain_gather(t, b)

                @pl.when(t + _NBUF - 1 < units_per_w)
                def _():
                    fire_gather(t + _NBUF - 1, (b + _NBUF - 1) % _NBUF)

                s, bb = unit_coords(t)

                @pl.when(t >= _NBUF)
                def _():
                    for jb in range(n_jb):
                        pltpu.make_async_copy(
                            tiles[b].at[pl.ds(jb * tile_words, tile_words)],
                            out_hbm.at[s, jb, bb],
                            wsems[b],
                        ).wait()

                def row_body(r2, c2):
                    for u_ in range(_UNROLL):
                        r = r2 * _UNROLL + u_
                        for k in range(d_model // 16):
                            vals = rows[b][r, pl.ds(k * 16, 16)] * SCALE
                            plsc.store_scatter(tiles[b], [jbase[k] + r], vals)
                    return c2

                lax.fori_loop(0, _LANES // _UNROLL, row_body, 0)

                for jb in range(n_jb):
                    pltpu.async_copy(
                        tiles[b].at[pl.ds(jb * tile_words, tile_words)],
                        out_hbm.at[s, jb, bb],
                        wsems[b],
                    )
            return carry

        for b in range(_NBUF - 1):
            fire_gather(b, b)
        lax.fori_loop(0, units_per_w // _NBUF, outer, 0)

        for b in range(_NBUF):
            for jb in range(n_jb):
                pltpu.make_async_copy(
                    tiles[b].at[pl.ds(jb * tile_words, tile_words)],
                    out_hbm.at[0, jb, 0],
                    wsems[b],
                ).wait()

    return pl.kernel(
        body,
        out_type=jax.ShapeDtypeStruct((n_seq, n_jb, n_bb, tile_words),
                                      jnp.float32),
        scratch_types=[
            pltpu.VMEM((units_per_w, _LANES), jnp.int32),
            [pltpu.VMEM((_LANES, D_MODEL), jnp.float32)
             for _ in range(_NBUF)],
            [pltpu.VMEM((n_jb * tile_words,), jnp.float32)
             for _ in range(_NBUF)],
            [pltpu.SemaphoreType.DMA for _ in range(_NBUF)],
            [pltpu.SemaphoreType.DMA for _ in range(_NBUF)],
        ],
        mesh=plsc.VectorSubcoreMesh(core_axis_name="c", subcore_axis_name="s"),
        compiler_params=pltpu.CompilerParams(use_tc_tiling_on_sc=False,
                                             needs_layout_passes=False),
    )


def kernel(x, table):
    n_batch, n_seq = x.shape
    vocab, d_model = table.shape
    n_bb = n_batch // _LANES
    n_jb = d_model // _SUB
    x4 = (x.T.astype(jnp.int32)
          .reshape(n_seq // _SUB, _SUB, n_bb, _LANES)
          .transpose(0, 2, 1, 3)
          .reshape(n_seq * n_bb, _LANES))
    out5 = _build(n_seq, n_batch, vocab, d_model)(x4, table)
    return (out5.reshape(n_seq, n_jb, n_bb, _SUB, _LANES)
            .transpose(2, 4, 0, 1, 3)
            .reshape(n_batch, n_seq, d_model))


# speedup vs baseline: 1.1164x; 1.1164x over previous
import functools
import math

import jax
import jax.numpy as jnp
from jax import lax
from jax.experimental import pallas as pl
from jax.experimental.pallas import tpu as pltpu
from jax.experimental.pallas import tpu_sc as plsc

D_MODEL = 64
SCALE = math.sqrt(D_MODEL)

_NC = 2
_NS = 16
_NW = _NC * _NS

_LANES = 128
_SUB = 8
_NBUF = 4
_UNROLL = 4


@functools.lru_cache(maxsize=None)
def _build(n_seq: int, n_batch: int, vocab: int, d_model: int):
    assert d_model == D_MODEL and n_batch % _LANES == 0 and n_seq % _SUB == 0
    n_bb = n_batch // _LANES
    n_units = n_seq * n_bb
    units_per_w = n_units // _NW
    assert n_units % (_NW * _NBUF) == 0
    n_jb = d_model // _SUB
    tile_words = _SUB * _LANES

    def body(x_hbm, table_hbm, out_hbm, idx_v, rows, tiles, gsems, wsems):
        wid = lax.axis_index("s") * _NC + lax.axis_index("c")
        u0 = wid * units_per_w

        pltpu.sync_copy(x_hbm.at[pl.ds(u0, units_per_w)], idx_v)

        iota = lax.iota(jnp.int32, 16)
        jrow = [(iota + 16 * k) // _SUB for k in range(d_model // 16)]
        jcol = [((iota + 16 * k) % _SUB) * _LANES
                for k in range(d_model // 16)]

        def fire_gather(t, slot):
            pltpu.async_copy(table_hbm.at[idx_v.at[t]], rows[slot],
                             gsems[slot])

        def drain_gather(t, slot):
            pltpu.make_async_copy(table_hbm.at[idx_v.at[t]], rows[slot],
                                  gsems[slot]).wait()

        def unit_coords(t):
            u = u0 + t
            s = (u >> 8) * _SUB + (u & (_SUB - 1))
            bb = (u >> 3) & (n_bb - 1)
            return s, bb

        def outer(o, carry):
            for b in range(_NBUF):
                t = o * _NBUF + b
                drain_gather(t, b)

                @pl.when(t + _NBUF - 1 < units_per_w)
                def _():
                    fire_gather(t + _NBUF - 1, (b + _NBUF - 1) % _NBUF)

                s, bb = unit_coords(t)

                @pl.when(t >= _NBUF)
                def _():
                    pltpu.make_async_copy(
                        tiles[b], out_hbm.at[s, :, bb], wsems[b],
                    ).wait()

                def row_body(r2, c2):
                    r0 = r2 * _UNROLL
                    nk = d_model // 16
                    vals = [
                        rows[b][r0 + u_, pl.ds(k * 16, 16)] * SCALE
                        for u_ in range(_UNROLL)
                        for k in range(nk)
                    ]
                    cols = [jcol[k] + (r0 + u_)
                            for u_ in range(_UNROLL)
                            for k in range(nk)]
                    for u_ in range(_UNROLL):
                        for k in range(nk):
                            plsc.store_scatter(
                                tiles[b], [jrow[k], cols[u_ * nk + k]],
                                vals[u_ * nk + k])
                    return c2

                lax.fori_loop(0, _LANES // _UNROLL, row_body, 0)

                pltpu.async_copy(tiles[b], out_hbm.at[s, :, bb], wsems[b])
            return carry

        for b in range(_NBUF - 1):
            fire_gather(b, b)
        lax.fori_loop(0, units_per_w // _NBUF, outer, 0)

        for b in range(_NBUF):
            pltpu.make_async_copy(tiles[b], out_hbm.at[0, :, 0],
                                  wsems[b]).wait()

    return pl.kernel(
        body,
        out_type=jax.ShapeDtypeStruct((n_seq, n_jb, n_bb, tile_words),
                                      jnp.float32),
        scratch_types=[
            pltpu.VMEM((units_per_w, _LANES), jnp.int32),
            [pltpu.VMEM((_LANES, D_MODEL), jnp.float32)
             for _ in range(_NBUF)],
            [pltpu.VMEM((n_jb, tile_words), jnp.float32)
             for _ in range(_NBUF)],
            [pltpu.SemaphoreType.DMA for _ in range(_NBUF)],
            [pltpu.SemaphoreType.DMA for _ in range(_NBUF)],
        ],
        mesh=plsc.VectorSubcoreMesh(core_axis_name="c", subcore_axis_name="s"),
        compiler_params=pltpu.CompilerParams(use_tc_tiling_on_sc=False,
                                             needs_layout_passes=False),
    )


def kernel(x, table):
    n_batch, n_seq = x.shape
    vocab, d_model = table.shape
    n_bb = n_batch // _LANES
    n_jb = d_model // _SUB
    x4 = (x.T.astype(jnp.int32)
          .reshape(n_seq // _SUB, _SUB, n_bb, _LANES)
          .transpose(0, 2, 1, 3)
          .reshape(n_seq * n_bb, _LANES))
    out5 = _build(n_seq, n_batch, vocab, d_model)(x4, table)
    return (out5.reshape(n_seq, n_jb, n_bb, _SUB, _LANES)
            .transpose(2, 4, 0, 1, 3)
            .reshape(n_batch, n_seq, d_model))


# speedup vs baseline: 1.7116x; 1.5331x over previous
import functools
import math

import jax
import jax.numpy as jnp
from jax import lax
from jax.experimental import pallas as pl
from jax.experimental.pallas import tpu as pltpu
from jax.experimental.pallas import tpu_sc as plsc

D_MODEL = 64
SCALE = math.sqrt(D_MODEL)

_NC = 2
_NS = 16
_NW = _NC * _NS

_LANES = 128
_SUB = 8
_NBUF = 4
_UNROLL = 4


@functools.lru_cache(maxsize=None)
def _build(n_seq: int, n_batch: int, vocab: int, d_model: int):
    assert d_model == D_MODEL and n_batch % _LANES == 0 and n_seq % _SUB == 0
    n_bb = n_batch // _LANES
    n_units = n_seq * n_bb
    units_per_w = n_units // _NW
    assert n_units % (_NW * _NBUF) == 0
    n_jb = d_model // _SUB
    tile_words = _SUB * _LANES

    def body(x_hbm, table_hbm, out_hbm, idx_v, rows, tiles, pad_v,
             gsems, wsems):
        wid = lax.axis_index("s") * _NC + lax.axis_index("c")
        u0 = wid * units_per_w

        pltpu.sync_copy(x_hbm.at[pl.ds(u0, units_per_w)], idx_v)

        iota = lax.iota(jnp.int32, 16)
        ridx = [iota + 16 * r2 for r2 in range(_LANES // 16)]

        def fire_gather(t, slot):
            pltpu.async_copy(table_hbm.at[idx_v.at[t]], rows[slot],
                             gsems[slot])

        def drain_gather(t, slot):
            pltpu.make_async_copy(table_hbm.at[idx_v.at[t]], rows[slot],
                                  gsems[slot]).wait()

        def unit_coords(t):
            u = u0 + t
            s = (u >> 8) * _SUB + (u & (_SUB - 1))
            bb = (u >> 3) & (n_bb - 1)
            return s, bb

        def outer(o, carry):
            for b in range(_NBUF):
                t = o * _NBUF + b
                drain_gather(t, b)

                @pl.when(t + _NBUF - 1 < units_per_w)
                def _():
                    fire_gather(t + _NBUF - 1, (b + _NBUF - 1) % _NBUF)

                s, bb = unit_coords(t)

                @pl.when(t >= _NBUF)
                def _():
                    pltpu.make_async_copy(
                        tiles[b], out_hbm.at[s, :, bb], wsems[b],
                    ).wait()

                nk = d_model // 16

                def pad_body(r3, c2):
                    vals = [
                        rows[b][r3 * 4 + u_, pl.ds(k * 16, 16)] * SCALE
                        for u_ in range(4) for k in range(nk)
                    ]
                    for u_ in range(4):
                        for k in range(nk):
                            pad_v[r3 * 4 + u_, pl.ds(k * 16, 16)] = (
                                vals[u_ * nk + k])
                    return c2

                lax.fori_loop(0, _LANES // 4, pad_body, 0)

                def jb_body(jb, c2):
                    for j8 in range(_SUB):
                        jsplat = jnp.full((16,), jb * _SUB + j8, jnp.int32)
                        vals = [
                            plsc.load_gather(pad_v, [ridx[r2], jsplat])
                            for r2 in range(_LANES // 16)
                        ]
                        for r2 in range(_LANES // 16):
                            tiles[b][jb, pl.ds(j8 * _LANES + r2 * 16,
                                               16)] = vals[r2]
                    return c2

                lax.fori_loop(0, n_jb, jb_body, 0)

                pltpu.async_copy(tiles[b], out_hbm.at[s, :, bb], wsems[b])
            return carry

        for b in range(_NBUF - 1):
            fire_gather(b, b)
        lax.fori_loop(0, units_per_w // _NBUF, outer, 0)

        for b in range(_NBUF):
            pltpu.make_async_copy(tiles[b], out_hbm.at[0, :, 0],
                                  wsems[b]).wait()

    return pl.kernel(
        body,
        out_type=jax.ShapeDtypeStruct((n_seq, n_jb, n_bb, tile_words),
                                      jnp.float32),
        scratch_types=[
            pltpu.VMEM((units_per_w, _LANES), jnp.int32),
            [pltpu.VMEM((_LANES, D_MODEL), jnp.float32)
             for _ in range(_NBUF)],
            [pltpu.VMEM((n_jb, tile_words), jnp.float32)
             for _ in range(_NBUF)],
            pltpu.VMEM((_LANES, D_MODEL + 1), jnp.float32),
            [pltpu.SemaphoreType.DMA for _ in range(_NBUF)],
            [pltpu.SemaphoreType.DMA for _ in range(_NBUF)],
        ],
        mesh=plsc.VectorSubcoreMesh(core_axis_name="c", subcore_axis_name="s"),
        compiler_params=pltpu.CompilerParams(use_tc_tiling_on_sc=False,
                                             needs_layout_passes=False),
    )


def kernel(x, table):
    n_batch, n_seq = x.shape
    vocab, d_model = table.shape
    n_bb = n_batch // _LANES
    n_jb = d_model // _SUB
    x4 = (x.T.astype(jnp.int32)
          .reshape(n_seq // _SUB, _SUB, n_bb, _LANES)
          .transpose(0, 2, 1, 3)
          .reshape(n_seq * n_bb, _LANES))
    out5 = _build(n_seq, n_batch, vocab, d_model)(x4, table)
    return (out5.reshape(n_seq, n_jb, n_bb, _SUB, _LANES)
            .transpose(2, 4, 0, 1, 3)
            .reshape(n_batch, n_seq, d_model))


# speedup vs baseline: 1.7786x; 1.0391x over previous
import functools
import math

import jax
import jax.numpy as jnp
from jax import lax
from jax.experimental import pallas as pl
from jax.experimental.pallas import tpu as pltpu
from jax.experimental.pallas import tpu_sc as plsc

D_MODEL = 64
SCALE = math.sqrt(D_MODEL)

_NC = 2
_NS = 16
_NW = _NC * _NS

_LANES = 128
_SUB = 8
_NBUF = 2
_PSTRIDE = _LANES + 1


@functools.lru_cache(maxsize=None)
def _build(n_seq: int, n_batch: int, vocab: int, d_model: int):
    assert d_model == D_MODEL and n_batch % _LANES == 0 and n_seq % _SUB == 0
    n_bb = n_batch // _LANES
    n_units = n_seq * n_bb
    units_per_w = n_units // _NW
    assert n_units % (_NW * _NBUF) == 0
    n_jb = d_model // _SUB
    nk = d_model // 16

    def body(x_hbm, table_hbm, out_hbm, idx_v, idx2, rows, tiles, pad_v,
             gsems, wsems):
        wid = lax.axis_index("s") * _NC + lax.axis_index("c")
        u0 = wid * units_per_w

        pltpu.sync_copy(x_hbm.at[pl.ds(u0, units_per_w)], idx_v)

        iota = lax.iota(jnp.int32, 16)
        rbase = [(iota + 16 * r2) * _PSTRIDE for r2 in range(_LANES // 16)]

        def fire_gather(t, slot):
            for r2 in range(_LANES // 16):
                idx2[slot][pl.ds(r2 * 16, 16)] = (
                    idx_v[t, pl.ds(r2 * 16, 16)] >> 1)
            pltpu.async_copy(table_hbm.at[idx2[slot]], rows[slot],
                             gsems[slot])

        def drain_gather(slot):
            pltpu.make_async_copy(table_hbm.at[idx2[slot]], rows[slot],
                                  gsems[slot]).wait()

        def unit_coords(t):
            u = u0 + t
            s = (u >> 8) * _SUB + (u & (_SUB - 1))
            bb = (u >> 3) & (n_bb - 1)
            return s, bb

        def outer(o, carry):
            for b in range(_NBUF):
                t = o * _NBUF + b
                drain_gather(b)

                @pl.when(t + _NBUF - 1 < units_per_w)
                def _():
                    fire_gather(t + _NBUF - 1, (b + _NBUF - 1) % _NBUF)

                s, bb = unit_coords(t)

                def pad_body(r3, c2):
                    vals = [
                        rows[b][r3 * 2 + u_, pl.ds(k * 16, 16)]
                        for u_ in range(2) for k in range(2 * nk)
                    ]
                    for u_ in range(2):
                        base = (r3 * 2 + u_) * _PSTRIDE
                        for k in range(2 * nk):
                            pad_v[pl.ds(base + k * 16, 16)] = (
                                vals[u_ * 2 * nk + k])
                    return c2

                lax.fori_loop(0, _LANES // 2, pad_body, 0)

                @pl.when(t >= _NBUF)
                def _():
                    pltpu.make_async_copy(
                        tiles[b], out_hbm.at[s, :, bb], wsems[b],
                    ).wait()

                def jb_body(jb, c2):
                    lsel = [
                        rbase[r2]
                        + ((idx_v[t, pl.ds(r2 * 16, 16)] & 1) << 6)
                        for r2 in range(_LANES // 16)
                    ]
                    for j8 in range(_SUB):
                        j = jb * _SUB + j8
                        vals = [
                            plsc.load_gather(pad_v, [lsel[r2] + j]) * SCALE
                            for r2 in range(_LANES // 16)
                        ]
                        for r2 in range(_LANES // 16):
                            tiles[b][jb, j8, pl.ds(r2 * 16, 16)] = vals[r2]
                    return c2

                lax.fori_loop(0, n_jb, jb_body, 0)

                pltpu.async_copy(tiles[b], out_hbm.at[s, :, bb], wsems[b])
            return carry

        for b in range(_NBUF - 1):
            fire_gather(b, b)
        lax.fori_loop(0, units_per_w // _NBUF, outer, 0)

        for b in range(_NBUF):
            pltpu.make_async_copy(tiles[b], out_hbm.at[0, :, 0],
                                  wsems[b]).wait()

    return pl.kernel(
        body,
        out_type=jax.ShapeDtypeStruct((n_seq, n_jb, n_bb, _SUB, _LANES),
                                      jnp.float32),
        scratch_types=[
            pltpu.VMEM((units_per_w, _LANES), jnp.int32),
            [pltpu.VMEM((_LANES,), jnp.int32) for _ in range(_NBUF)],
            [pltpu.VMEM((_LANES, 2 * D_MODEL), jnp.float32)
             for _ in range(_NBUF)],
            [pltpu.VMEM((n_jb, _SUB, _LANES), jnp.float32)
             for _ in range(_NBUF)],
            pltpu.VMEM((_LANES * _PSTRIDE,), jnp.float32),
            [pltpu.SemaphoreType.DMA for _ in range(_NBUF)],
            [pltpu.SemaphoreType.DMA for _ in range(_NBUF)],
        ],
        mesh=plsc.VectorSubcoreMesh(core_axis_name="c", subcore_axis_name="s"),
        compiler_params=pltpu.CompilerParams(use_tc_tiling_on_sc=True,
                                             needs_layout_passes=False),
    )


def kernel(x, table):
    n_batch, n_seq = x.shape
    vocab, d_model = table.shape
    n_bb = n_batch // _LANES
    n_jb = d_model // _SUB
    x4 = (x.T.astype(jnp.int32)
          .reshape(n_seq // _SUB, _SUB, n_bb, _LANES)
          .transpose(0, 2, 1, 3)
          .reshape(n_seq * n_bb, _LANES))
    tblr = table.reshape(vocab // 2, 2 * d_model)
    out5 = _build(n_seq, n_batch, vocab, d_model)(x4, tblr)
    return (out5.transpose(2, 4, 0, 1, 3)
            .reshape(n_batch, n_seq, d_model))


# speedup vs baseline: 2.4285x; 1.3654x over previous
import functools
import math

import jax
import jax.numpy as jnp
from jax import lax
from jax.experimental import pallas as pl
from jax.experimental.pallas import tpu as pltpu
from jax.experimental.pallas import tpu_sc as plsc

D_MODEL = 64
SCALE = math.sqrt(D_MODEL)

_NC = 2
_NS = 16
_NW = _NC * _NS

_LANES = 128
_SUB = 8
_NBUF = 2
_ABUF = 2

_params = pltpu.CompilerParams(use_tc_tiling_on_sc=True,
                               needs_layout_passes=False)


@functools.lru_cache(maxsize=None)
def _build_relayout(vocab: int, d_model: int):
    n_full = vocab // _LANES
    tail = vocab % _LANES
    nk = d_model // 16
    base_n = n_full // _NW
    rem = n_full % _NW
    pstride = 2 * d_model + 1

    def body(tp_hbm, tail_hbm, out_hbm, slabs, outs, pad_v, gsems, wsems):
        wid = lax.axis_index("s") * _NC + lax.axis_index("c")
        n_i = base_n + jnp.where(wid < rem, 1, 0)

        iota = lax.iota(jnp.int32, 16)
        cb = [(iota + 16 * c) * pstride for c in range(nk)]

        def vb_of(i):
            return wid + _NW * i

        def fire(i, slot):
            pltpu.async_copy(
                tp_hbm.at[:, pl.ds(vb_of(i) * _LANES, _LANES)],
                slabs[slot], gsems[slot])

        def drain(i, slot):
            pltpu.make_async_copy(
                tp_hbm.at[:, pl.ds(vb_of(i) * _LANES, _LANES)],
                slabs[slot], gsems[slot]).wait()

        def transpose_write(i, b, out_rows, out_row0):
            def p1(q, c2):
                vals = [
                    slabs[b][q * 2 + u_, pl.ds(c * 16, 16)] * SCALE
                    for u_ in range(2) for c in range(2 * nk)
                ]
                for u_ in range(2):
                    base = (q * 2 + u_) * pstride
                    for c in range(2 * nk):
                        pad_v[pl.ds(base + c * 16, 16)] = (
                            vals[u_ * 2 * nk + c])
                return c2

            lax.fori_loop(0, d_model // 2, p1, 0)

            @pl.when(i >= _ABUF)
            def _():
                pltpu.make_async_copy(
                    outs[b].at[pl.ds(0, _LANES // 2)],
                    out_hbm.at[pl.ds(0, _LANES // 2)], wsems[b]).wait()

            def p2(v2, c2):
                for u_ in range(4):
                    v = v2 * 4 + u_
                    vals = [plsc.load_gather(pad_v, [cb[c] + v])
                            for c in range(nk)]
                    p, off = v >> 1, (v & 1) * d_model
                    for c in range(nk):
                        outs[b][p, pl.ds(off + c * 16, 16)] = vals[c]
                return c2

            lax.fori_loop(0, 2 * out_rows // 4, p2, 0)

            pltpu.async_copy(
                outs[b].at[pl.ds(0, out_rows)],
                out_hbm.at[pl.ds(out_row0, out_rows)],
                wsems[b])

        def process(i, b):
            drain(i, b)

            @pl.when(i + 1 < n_i)
            def _():
                fire(i + 1, 1 - b)

            transpose_write(i, b, _LANES // 2, vb_of(i) * (_LANES // 2))

        fire(0, 0)

        def loop_body(o, carry):
            for b in range(_ABUF):
                process(o * _ABUF + b, b)
            return carry

        lax.fori_loop(0, base_n // _ABUF, loop_body, 0)

        @pl.when(wid < rem)
        def _():
            process(base_n, 0)

        @pl.when(wid == rem)
        def _():
            pltpu.async_copy(tail_hbm, slabs[0], gsems[0])
            pltpu.make_async_copy(tail_hbm, slabs[0], gsems[0]).wait()
            transpose_write(base_n, 0, tail // 2,
                            (vocab - tail) // 2)

        @pl.when(wid == rem)
        def _():
            pltpu.make_async_copy(outs[0].at[pl.ds(0, tail // 2)],
                                  out_hbm.at[pl.ds(0, tail // 2)],
                                  wsems[0]).wait()

        @pl.when(wid != rem)
        def _():
            pltpu.make_async_copy(outs[0].at[pl.ds(0, _LANES // 2)],
                                  out_hbm.at[pl.ds(0, _LANES // 2)],
                                  wsems[0]).wait()

        pltpu.make_async_copy(outs[1].at[pl.ds(0, _LANES // 2)],
                              out_hbm.at[pl.ds(0, _LANES // 2)],
                              wsems[1]).wait()

    return pl.kernel(
        body,
        out_type=jax.ShapeDtypeStruct((vocab // 2, 2 * d_model),
                                      jnp.float32),
        scratch_types=[
            [pltpu.VMEM((d_model, _LANES), jnp.float32)
             for _ in range(_ABUF)],
            [pltpu.VMEM((_LANES // 2, 2 * d_model), jnp.float32)
             for _ in range(_ABUF)],
            pltpu.VMEM((d_model * pstride,), jnp.float32),
            [pltpu.SemaphoreType.DMA for _ in range(_ABUF)],
            [pltpu.SemaphoreType.DMA for _ in range(_ABUF)],
        ],
        mesh=plsc.VectorSubcoreMesh(core_axis_name="c",
                                    subcore_axis_name="s"),
        compiler_params=_params,
    )


@functools.lru_cache(maxsize=None)
def _build_gather(n_seq: int, n_batch: int, vocab: int, d_model: int):
    n_bb = n_batch // _LANES
    n_units = n_seq * n_bb
    units_per_w = n_units // _NW
    assert n_units % (_NW * _NBUF) == 0
    n_jb = d_model // _SUB
    nk = d_model // 16
    pstride = _LANES + 1

    def body(x_hbm, table_hbm, out_hbm, idx_v, idx2, rows, tiles, pad_v,
             gsems, wsems):
        wid = lax.axis_index("s") * _NC + lax.axis_index("c")
        u0 = wid * units_per_w

        pltpu.sync_copy(x_hbm.at[pl.ds(u0, units_per_w)], idx_v)

        iota = lax.iota(jnp.int32, 16)
        rbase = [(iota + 16 * r2) * pstride for r2 in range(_LANES // 16)]

        def fire_gather(t, slot):
            for r2 in range(_LANES // 16):
                idx2[slot][pl.ds(r2 * 16, 16)] = (
                    idx_v[t, pl.ds(r2 * 16, 16)] >> 1)
            pltpu.async_copy(table_hbm.at[idx2[slot]], rows[slot],
                             gsems[slot])

        def drain_gather(slot):
            pltpu.make_async_copy(table_hbm.at[idx2[slot]], rows[slot],
                                  gsems[slot]).wait()

        def unit_coords(t):
            u = u0 + t
            s = (u >> 8) * _SUB + (u & (_SUB - 1))
            bb = (u >> 3) & (n_bb - 1)
            return s, bb

        def outer(o, carry):
            for b in range(_NBUF):
                t = o * _NBUF + b
                drain_gather(b)

                @pl.when(t + _NBUF - 1 < units_per_w)
                def _():
                    fire_gather(t + _NBUF - 1, (b + _NBUF - 1) % _NBUF)

                s, bb = unit_coords(t)

                def pad_body(r3, c2):
                    vals = [
                        rows[b][r3 * 2 + u_, pl.ds(k * 16, 16)]
                        for u_ in range(2) for k in range(2 * nk)
                    ]
                    for u_ in range(2):
                        base = (r3 * 2 + u_) * pstride
                        for k in range(2 * nk):
                            pad_v[pl.ds(base + k * 16, 16)] = (
                                vals[u_ * 2 * nk + k])
                    return c2

                lax.fori_loop(0, _LANES // 2, pad_body, 0)

                @pl.when(t >= _NBUF)
                def _():
                    pltpu.make_async_copy(
                        tiles[b], out_hbm.at[s, :, bb], wsems[b],
                    ).wait()

                def jb_body(jb, c2):
                    lsel = [
                        rbase[r2]
                        + ((idx_v[t, pl.ds(r2 * 16, 16)] & 1) << 6)
                        for r2 in range(_LANES // 16)
                    ]
                    for j8 in range(_SUB):
                        j = jb * _SUB + j8
                        vals = [
                            plsc.load_gather(pad_v, [lsel[r2] + j])
                            for r2 in range(_LANES // 16)
                        ]
                        for r2 in range(_LANES // 16):
                            tiles[b][jb, j8, pl.ds(r2 * 16, 16)] = vals[r2]
                    return c2

                lax.fori_loop(0, n_jb, jb_body, 0)

                pltpu.async_copy(tiles[b], out_hbm.at[s, :, bb], wsems[b])
            return carry

        for b in range(_NBUF - 1):
            fire_gather(b, b)
        lax.fori_loop(0, units_per_w // _NBUF, outer, 0)

        for b in range(_NBUF):
            pltpu.make_async_copy(tiles[b], out_hbm.at[0, :, 0],
                                  wsems[b]).wait()

    return pl.kernel(
        body,
        out_type=jax.ShapeDtypeStruct((n_seq, n_jb, n_bb, _SUB, _LANES),
                                      jnp.float32),
        scratch_types=[
            pltpu.VMEM((units_per_w, _LANES), jnp.int32),
            [pltpu.VMEM((_LANES,), jnp.int32) for _ in range(_NBUF)],
            [pltpu.VMEM((_LANES, 2 * D_MODEL), jnp.float32)
             for _ in range(_NBUF)],
            [pltpu.VMEM((n_jb, _SUB, _LANES), jnp.float32)
             for _ in range(_NBUF)],
            pltpu.VMEM((_LANES * pstride,), jnp.float32),
            [pltpu.SemaphoreType.DMA for _ in range(_NBUF)],
            [pltpu.SemaphoreType.DMA for _ in range(_NBUF)],
        ],
        mesh=plsc.VectorSubcoreMesh(core_axis_name="c",
                                    subcore_axis_name="s"),
        compiler_params=_params,
    )


def kernel(x, table):
    n_batch, n_seq = x.shape
    vocab, d_model = table.shape
    n_bb = n_batch // _LANES
    n_jb = d_model // _SUB
    x4 = (x.T.astype(jnp.int32)
          .reshape(n_seq // _SUB, _SUB, n_bb, _LANES)
          .transpose(0, 2, 1, 3)
          .reshape(n_seq * n_bb, _LANES))
    tp = table.T
    tail = vocab % _LANES
    tail_p = jnp.pad(tp[:, vocab - tail:], ((0, 0), (0, _LANES - tail)))
    table_rm = _build_relayout(vocab, d_model)(tp, tail_p)
    out5 = _build_gather(n_seq, n_batch, vocab, d_model)(x4, table_rm)
    return (out5.transpose(2, 4, 0, 1, 3)
            .reshape(n_batch, n_seq, d_model))


# speedup vs baseline: 2.9248x; 1.2043x over previous
import functools
import math

import jax
import jax.numpy as jnp
from jax import lax
from jax.experimental import pallas as pl
from jax.experimental.pallas import tpu as pltpu
from jax.experimental.pallas import tpu_sc as plsc

D_MODEL = 64
SCALE = math.sqrt(D_MODEL)

_NC = 2
_NS = 16
_NW = _NC * _NS

_LANES = 128
_SUB = 8
_NBUF = 4
_ABUF = 2

_params = pltpu.CompilerParams(use_tc_tiling_on_sc=True,
                               needs_layout_passes=False)


@functools.lru_cache(maxsize=None)
def _build_relayout(vocab: int, d_model: int):
    n_full = vocab // _LANES
    tail = vocab % _LANES
    nk = d_model // 16
    base_n = n_full // _NW
    rem = n_full % _NW
    pstride = 2 * d_model + 1

    def body(tp_hbm, tail_hbm, out_hbm, slabs, outs, pad_v, gsems, wsems):
        wid = lax.axis_index("s") * _NC + lax.axis_index("c")
        n_i = base_n + jnp.where(wid < rem, 1, 0)

        iota = lax.iota(jnp.int32, 16)
        cb = [(iota + 16 * c) * pstride for c in range(nk)]

        def vb_of(i):
            return wid + _NW * i

        def fire(i, slot):
            pltpu.async_copy(
                tp_hbm.at[:, pl.ds(vb_of(i) * _LANES, _LANES)],
                slabs[slot], gsems[slot])

        def drain(i, slot):
            pltpu.make_async_copy(
                tp_hbm.at[:, pl.ds(vb_of(i) * _LANES, _LANES)],
                slabs[slot], gsems[slot]).wait()

        def transpose_write(i, b, out_rows, out_row0):
            def p1(q, c2):
                vals = [
                    slabs[b][q * 2 + u_, pl.ds(c * 16, 16)] * SCALE
                    for u_ in range(2) for c in range(2 * nk)
                ]
                for u_ in range(2):
                    base = (q * 2 + u_) * pstride
                    for c in range(2 * nk):
                        pad_v[pl.ds(base + c * 16, 16)] = (
                            vals[u_ * 2 * nk + c])
                return c2

            lax.fori_loop(0, d_model // 2, p1, 0)

            @pl.when(i >= _ABUF)
            def _():
                pltpu.make_async_copy(
                    outs[b].at[pl.ds(0, _LANES // 2)],
                    out_hbm.at[pl.ds(0, _LANES // 2)], wsems[b]).wait()

            def p2(v2, c2):
                for u_ in range(4):
                    v = v2 * 4 + u_
                    vals = [plsc.load_gather(pad_v, [cb[c] + v])
                            for c in range(nk)]
                    p, off = v >> 1, (v & 1) * d_model
                    for c in range(nk):
                        outs[b][p, pl.ds(off + c * 16, 16)] = vals[c]
                return c2

            lax.fori_loop(0, 2 * out_rows // 4, p2, 0)

            pltpu.async_copy(
                outs[b].at[pl.ds(0, out_rows)],
                out_hbm.at[pl.ds(out_row0, out_rows)],
                wsems[b])

        def process(i, b):
            drain(i, b)

            @pl.when(i + 1 < n_i)
            def _():
                fire(i + 1, 1 - b)

            transpose_write(i, b, _LANES // 2, vb_of(i) * (_LANES // 2))

        fire(0, 0)

        def loop_body(o, carry):
            for b in range(_ABUF):
                process(o * _ABUF + b, b)
            return carry

        lax.fori_loop(0, base_n // _ABUF, loop_body, 0)

        @pl.when(wid < rem)
        def _():
            process(base_n, 0)

        @pl.when(wid == rem)
        def _():
            pltpu.async_copy(tail_hbm, slabs[0], gsems[0])
            pltpu.make_async_copy(tail_hbm, slabs[0], gsems[0]).wait()
            transpose_write(base_n, 0, tail // 2,
                            (vocab - tail) // 2)

        @pl.when(wid == rem)
        def _():
            pltpu.make_async_copy(outs[0].at[pl.ds(0, tail // 2)],
                                  out_hbm.at[pl.ds(0, tail // 2)],
                                  wsems[0]).wait()

        @pl.when(wid != rem)
        def _():
            pltpu.make_async_copy(outs[0].at[pl.ds(0, _LANES // 2)],
                                  out_hbm.at[pl.ds(0, _LANES // 2)],
                                  wsems[0]).wait()

        pltpu.make_async_copy(outs[1].at[pl.ds(0, _LANES // 2)],
                              out_hbm.at[pl.ds(0, _LANES // 2)],
                              wsems[1]).wait()

    return pl.kernel(
        body,
        out_type=jax.ShapeDtypeStruct((vocab // 2, 2 * d_model),
                                      jnp.float32),
        scratch_types=[
            [pltpu.VMEM((d_model, _LANES), jnp.float32)
             for _ in range(_ABUF)],
            [pltpu.VMEM((_LANES // 2, 2 * d_model), jnp.float32)
             for _ in range(_ABUF)],
            pltpu.VMEM((d_model * pstride,), jnp.float32),
            [pltpu.SemaphoreType.DMA for _ in range(_ABUF)],
            [pltpu.SemaphoreType.DMA for _ in range(_ABUF)],
        ],
        mesh=plsc.VectorSubcoreMesh(core_axis_name="c",
                                    subcore_axis_name="s"),
        compiler_params=_params,
    )


@functools.lru_cache(maxsize=None)
def _build_gather(n_seq: int, n_batch: int, vocab: int, d_model: int):
    n_bb = n_batch // _LANES
    n_units = n_seq * n_bb
    units_per_w = n_units // _NW
    assert n_units % (_NW * _NBUF) == 0
    n_jb = d_model // _SUB
    nk = d_model // 16
    pstride = d_model + 1

    def body(x_hbm, table_hbm, out_hbm, idx_v, rows, tiles, pad_v,
             gsems, wsems):
        wid = lax.axis_index("s") * _NC + lax.axis_index("c")
        u0 = wid * units_per_w

        pltpu.sync_copy(x_hbm.at[pl.ds(u0, units_per_w)], idx_v)

        iota = lax.iota(jnp.int32, 16)
        rbase = [(iota + 16 * r2) * pstride for r2 in range(_LANES // 16)]

        def fire_gather(t, slot):
            pltpu.async_copy(table_hbm.at[idx_v.at[t]], rows[slot],
                             gsems[slot])

        def drain_gather(t, slot):
            pltpu.make_async_copy(table_hbm.at[idx_v.at[t]], rows[slot],
                                  gsems[slot]).wait()

        def unit_coords(t):
            u = u0 + t
            s = (u >> 8) * _SUB + (u & (_SUB - 1))
            bb = (u >> 3) & (n_bb - 1)
            return s, bb

        def outer(o, carry):
            for b in range(_NBUF):
                t = o * _NBUF + b
                drain_gather(t, b)

                @pl.when(t + _NBUF - 1 < units_per_w)
                def _():
                    fire_gather(t + _NBUF - 1, (b + _NBUF - 1) % _NBUF)

                s, bb = unit_coords(t)

                def pad_body(r3, c2):
                    vals = [
                        rows[b][r3 * 4 + u_, pl.ds(k * 16, 16)]
                        for u_ in range(4) for k in range(nk)
                    ]
                    for u_ in range(4):
                        base = (r3 * 4 + u_) * pstride
                        for k in range(nk):
                            pad_v[pl.ds(base + k * 16, 16)] = (
                                vals[u_ * nk + k])
                    return c2

                lax.fori_loop(0, _LANES // 4, pad_body, 0)

                @pl.when(t >= _NBUF)
                def _():
                    pltpu.make_async_copy(
                        tiles[b], out_hbm.at[s, :, bb], wsems[b],
                    ).wait()

                def jb_body(jb, c2):
                    for j8 in range(_SUB):
                        j = jb * _SUB + j8
                        vals = [
                            plsc.load_gather(pad_v, [rbase[r2] + j])
                            for r2 in range(_LANES // 16)
                        ]
                        for r2 in range(_LANES // 16):
                            tiles[b][jb, j8, pl.ds(r2 * 16, 16)] = vals[r2]
                    return c2

                lax.fori_loop(0, n_jb, jb_body, 0)

                pltpu.async_copy(tiles[b], out_hbm.at[s, :, bb], wsems[b])
            return carry

        for b in range(_NBUF - 1):
            fire_gather(b, b)
        lax.fori_loop(0, units_per_w // _NBUF, outer, 0)

        for b in range(_NBUF):
            pltpu.make_async_copy(tiles[b], out_hbm.at[0, :, 0],
                                  wsems[b]).wait()

    return pl.kernel(
        body,
        out_type=jax.ShapeDtypeStruct((n_seq, n_jb, n_bb, _SUB, _LANES),
                                      jnp.float32),
        scratch_types=[
            pltpu.VMEM((units_per_w, _LANES), jnp.int32),
            [pltpu.VMEM((_LANES, D_MODEL), jnp.float32)
             for _ in range(_NBUF)],
            [pltpu.VMEM((n_jb, _SUB, _LANES), jnp.float32)
             for _ in range(_NBUF)],
            pltpu.VMEM((_LANES * pstride,), jnp.float32),
            [pltpu.SemaphoreType.DMA for _ in range(_NBUF)],
            [pltpu.SemaphoreType.DMA for _ in range(_NBUF)],
        ],
        mesh=plsc.VectorSubcoreMesh(core_axis_name="c",
                                    subcore_axis_name="s"),
        compiler_params=pltpu.CompilerParams(use_tc_tiling_on_sc=False,
                                             needs_layout_passes=False),
    )


def kernel(x, table):
    n_batch, n_seq = x.shape
    vocab, d_model = table.shape
    n_bb = n_batch // _LANES
    n_jb = d_model // _SUB
    x4 = (x.T.astype(jnp.int32)
          .reshape(n_seq // _SUB, _SUB, n_bb, _LANES)
          .transpose(0, 2, 1, 3)
          .reshape(n_seq * n_bb, _LANES))
    tp = table.T
    tail = vocab % _LANES
    tail_p = jnp.pad(tp[:, vocab - tail:], ((0, 0), (0, _LANES - tail)))
    table_rm = _build_relayout(vocab, d_model)(tp, tail_p)
    out5 = _build_gather(n_seq, n_batch, vocab, d_model)(
        x4, table_rm.reshape(vocab, d_model))
    return (out5.transpose(2, 4, 0, 1, 3)
            .reshape(n_batch, n_seq, d_model))
